# baseline (speedup 1.0000x reference)
import functools
import jax
import jax.numpy as jnp
from jax import lax
from jax.experimental import pallas as pl
from jax.experimental.pallas import tpu as pltpu
from jax.experimental.pallas import tpu_sc as plsc

NUSERS = 30000
NNODES = 50000
D = 64
NEDGES = 800000
B = 4096

NC = 2
NS = 16
NW = NC * NS

BINS = NW
BINR = 1568

CHP = 24
NBIGP = 17
PROWS_T = NBIGP * CHP
EPADP = NW * PROWS_T * 64
CAP = 1056

SUB = 48
KF = 2
NSUB = CAP // SUB
BPC = NSUB // KF
NPIECE = NW
TB = NPIECE * BPC

BPT = B // NW

_mesh = plsc.VectorSubcoreMesh(core_axis_name="c", subcore_axis_name="s")

_GATHER_DN = lax.GatherDimensionNumbers(
    offset_dims=(), collapsed_slice_dims=(0,), start_index_map=(0,))


def _bcast_lane(v16, lane):
    return lax.gather(v16, jnp.full((16, 1), lane, jnp.int32), _GATHER_DN,
                      slice_sizes=(1,),
                      mode=lax.GatherScatterMode.PROMISE_IN_BOUNDS)


def _shuffle(v16, idx16):
    return lax.gather(v16, idx16[:, None], _GATHER_DN, slice_sizes=(1,),
                      mode=lax.GatherScatterMode.PROMISE_IN_BOUNDS)


def _lane_reduce_sum(v16, lanes):
    for sh in (8, 4, 2, 1):
        v16 = v16 + _shuffle(v16, lanes ^ sh)
    return v16


@functools.partial(
    pl.kernel,
    out_type=[
        jax.ShapeDtypeStruct((NW, BINS * CAP), jnp.int32),
        jax.ShapeDtypeStruct((NW, BINS * CAP), jnp.float32),
        jax.ShapeDtypeStruct((NW, BINS * CAP), jnp.int32),
    ],
    mesh=_mesh,
    compiler_params=pltpu.CompilerParams(use_tc_tiling_on_sc=False,
                                         needs_layout_passes=False),
    scratch_types=[
        pltpu.VMEM((2, CHP, 64), jnp.int32),
        pltpu.VMEM((2, CHP, 64), jnp.int32),
        pltpu.VMEM((2, CHP, 64), jnp.float32),
        pltpu.VMEM((BINS * CAP,), jnp.int32),
        pltpu.VMEM((BINS * CAP,), jnp.float32),
        pltpu.VMEM((BINS * CAP,), jnp.int32),
        pltpu.VMEM((32,), jnp.int32),
        pltpu.SemaphoreType.DMA,
    ],
)
def _partition(row_hbm, col_hbm, val_hbm, colp_hbm, valp_hbm, lrowp_hbm,
               row_v, col_v, val_v, oc, ov, olr, cnt_v, sem_st):
    c = lax.axis_index("c")
    s = lax.axis_index("s")
    w = s * NC + c
    lanes = lax.iota(jnp.int32, 16)
    m0 = lanes == 0
    izero = jnp.zeros((16,), jnp.int32)
    fzero = jnp.zeros((16,), jnp.float32)

    cnt_v[pl.ds(0, 16)] = izero
    cnt_v[pl.ds(16, 16)] = izero

    def zi(i, carry):
        oc[pl.ds(i * 16, 16)] = izero
        ov[pl.ds(i * 16, 16)] = fzero
        olr[pl.ds(i * 16, 16)] = izero
        return carry
    lax.fori_loop(0, BINS * CAP // 16, zi, 0)

    sbase = w * PROWS_T

    def stage_fire(ck, parity):
        off = sbase + ck * CHP
        pltpu.async_copy(row_hbm.at[pl.ds(off, CHP)], row_v.at[parity],
                         sem_st)
        pltpu.async_copy(col_hbm.at[pl.ds(off, CHP)], col_v.at[parity],
                         sem_st)
        pltpu.async_copy(val_hbm.at[pl.ds(off, CHP)], val_v.at[parity],
                         sem_st)

    def stage_drain():
        pltpu.make_async_copy(row_hbm.at[pl.ds(0, CHP)], row_v.at[0],
                              sem_st).wait()
        pltpu.make_async_copy(col_hbm.at[pl.ds(0, CHP)], col_v.at[0],
                              sem_st).wait()
        pltpu.make_async_copy(val_hbm.at[pl.ds(0, CHP)], val_v.at[0],
                              sem_st).wait()

    stage_fire(0, 0)

    def chunk_iter(ck, carry):
        parity = lax.rem(ck, 2)
        stage_drain()

        @pl.when(ck + 1 <= NBIGP - 1)
        def _pf():
            stage_fire(ck + 1, lax.rem(ck + 1, 2))

        def row_iter(r, carry2):
            for q in range(4):
                rv = row_v[parity, r, pl.ds(q * 16, 16)]
                cv = col_v[parity, r, pl.ds(q * 16, 16)]
                vv = val_v[parity, r, pl.ds(q * 16, 16)]
                binv = rv // BINR
                lrv = rv - binv * BINR
                for l in range(16):
                    bb = _bcast_lane(binv, l)
                    base = jnp.minimum(plsc.load_gather(cnt_v, [bb]), CAP - 1)
                    dest = bb * CAP + base
                    plsc.store_scatter(oc, [dest], _bcast_lane(cv, l),
                                       mask=m0)
                    plsc.store_scatter(ov, [dest], _bcast_lane(vv, l),
                                       mask=m0)
                    plsc.store_scatter(olr, [dest], _bcast_lane(lrv, l),
                                       mask=m0)
                    plsc.store_scatter(cnt_v, [bb], base + 1, mask=m0)
            return carry2
        lax.fori_loop(0, CHP, row_iter, 0)
        return carry

    lax.fori_loop(0, NBIGP, chunk_iter, 0)

    pltpu.sync_copy(oc, colp_hbm.at[w])
    pltpu.sync_copy(ov, valp_hbm.at[w])
    pltpu.sync_copy(olr, lrowp_hbm.at[w])


@functools.partial(
    pl.kernel,
    out_type=jax.ShapeDtypeStruct((NNODES, D), jnp.float32),
    mesh=_mesh,
    compiler_params=pltpu.CompilerParams(use_tc_tiling_on_sc=False,
                                         needs_layout_passes=False),
    scratch_types=[
        pltpu.VMEM((2, NSUB, SUB), jnp.int32),
        pltpu.VMEM((2, NSUB, SUB), jnp.float32),
        pltpu.VMEM((2, NSUB, SUB), jnp.int32),
        pltpu.VMEM((3 * KF, SUB, D), jnp.float32),
        pltpu.VMEM((BINR, D), jnp.float32),
        pltpu.SemaphoreType.DMA,
        pltpu.SemaphoreType.DMA,
    ],
)
def _spmm(x_hbm, colp_hbm, valp_hbm, lrowp_hbm, y_hbm,
          col_v, val_v, lrow_v, g_v, acc, sem_g, sem_st):
    c = lax.axis_index("c")
    s = lax.axis_index("s")
    w = s * NC + c
    lanes = lax.iota(jnp.int32, 16)
    cols_q = [lanes + 16 * q for q in range(D // 16)]
    fzero = jnp.zeros((16,), jnp.float32)

    def zr(i, carry):
        for q in range(D // 16):
            acc[i, pl.ds(q * 16, 16)] = fzero
        return carry
    lax.fori_loop(0, BINR, zr, 0)

    def stage_fire(pc_idx, parity):
        pltpu.async_copy(colp_hbm.at[pc_idx, w], col_v.at[parity], sem_st)
        pltpu.async_copy(valp_hbm.at[pc_idx, w], val_v.at[parity], sem_st)
        pltpu.async_copy(lrowp_hbm.at[pc_idx, w], lrow_v.at[parity], sem_st)

    def stage_drain():
        pltpu.make_async_copy(colp_hbm.at[0, 0], col_v.at[0], sem_st).wait()
        pltpu.make_async_copy(valp_hbm.at[0, 0], val_v.at[0], sem_st).wait()
        pltpu.make_async_copy(lrowp_hbm.at[0, 0], lrow_v.at[0], sem_st).wait()

    def fire_gathers(t, slot_base):
        pcf = lax.rem(t // BPC, 2)
        jo = lax.rem(t, BPC) * KF
        for k in range(KF):
            pltpu.async_copy(x_hbm.at[col_v.at[pcf, jo + k]],
                             g_v.at[slot_base + k], sem_g)

    def drain_gathers(slot_base):
        for k in range(KF):
            pltpu.make_async_copy(x_hbm.at[pl.ds(0, SUB)],
                                  g_v.at[slot_base + k], sem_g).wait()

    def compute_batch(t, slot_base):
        pc = lax.rem(t // BPC, 2)
        jo = lax.rem(t, BPC) * KF
        for k in range(KF):
            def grp(gi2, carry):
                lr16 = lrow_v[pc, jo + k, pl.ds(gi2 * 16, 16)]
                vv16 = val_v[pc, jo + k, pl.ds(gi2 * 16, 16)]
                for l in range(16):
                    lrb = _bcast_lane(lr16, l)
                    vvb = _bcast_lane(vv16, l)
                    e = gi2 * 16 + l
                    for q in range(D // 16):
                        plsc.addupdate_scatter(
                            acc, [lrb, cols_q[q]],
                            g_v[slot_base + k, e, pl.ds(q * 16, 16)] * vvb)
                return carry
            lax.fori_loop(0, SUB // 16, grp, 0)

    def body(t, set_x, fire_g):
        drain_gathers(set_x)

        @pl.when(lax.rem(t, BPC) == 0)
        def _pf():
            cc = t // BPC

            @pl.when(cc + 1 <= NPIECE - 1)
            def _fire():
                stage_fire(cc + 1, lax.rem(cc + 1, 2))

        @pl.when((lax.rem(t, BPC) == BPC - 2) & (t < TB - 2))
        def _drain():
            stage_drain()

        if fire_g:
            set_f = ((t + 2) % 3) * KF
            fire_gathers(t + 2, set_f)
        compute_batch(t, set_x)

    stage_fire(0, 0)
    stage_drain()
    fire_gathers(0, 0)
    fire_gathers(1, KF)

    body(0, 0, True)
    body(1, KF, True)

    def triple(tt, carry):
        t0 = 2 + tt * 3
        body(t0, (t0 % 3) * KF, True)
        body(t0 + 1, ((t0 + 1) % 3) * KF, True)
        body(t0 + 2, ((t0 + 2) % 3) * KF, True)
        return carry
    lax.fori_loop(0, (TB - 4) // 3, triple, 0)

    body(TB - 2, ((TB - 2) % 3) * KF, False)
    body(TB - 1, ((TB - 1) % 3) * KF, False)

    @pl.when(w < NW - 1)
    def _wb():
        pltpu.sync_copy(acc, y_hbm.at[pl.ds(w * BINR, BINR)])

    @pl.when(w == NW - 1)
    def _wb_last():
        tail = NNODES - (NW - 1) * BINR
        pltpu.sync_copy(acc.at[pl.ds(0, tail)],
                        y_hbm.at[pl.ds((NW - 1) * BINR, tail)])


@functools.partial(
    pl.kernel,
    out_type=jax.ShapeDtypeStruct((B,), jnp.float32),
    mesh=_mesh,
    compiler_params=pltpu.CompilerParams(use_tc_tiling_on_sc=False),
    scratch_types=[
        pltpu.VMEM((BPT,), jnp.int32),
        pltpu.VMEM((BPT,), jnp.int32),
        pltpu.VMEM((4, BPT, D), jnp.float32),
        pltpu.VMEM((4, BPT, D), jnp.float32),
        pltpu.VMEM((BPT,), jnp.float32),
        pltpu.SemaphoreType.DMA,
    ],
)
def _score(x0, x1, x2, x3, ui_hbm, ii_hbm, out_hbm,
           ub, ib, gu, gi, ob, sem):
    c = lax.axis_index("c")
    s = lax.axis_index("s")
    w = s * NC + c
    base = w * BPT

    pltpu.sync_copy(ui_hbm.at[pl.ds(base, BPT)], ub)
    pltpu.sync_copy(ii_hbm.at[pl.ds(base, BPT)], ib)
    for t, x in enumerate((x0, x1, x2, x3)):
        pltpu.async_copy(x.at[ub], gu.at[t], sem)
        pltpu.async_copy(x.at[ib], gi.at[t], sem)
    for t, x in enumerate((x0, x1, x2, x3)):
        pltpu.make_async_copy(x.at[pl.ds(0, BPT)], gu.at[t], sem).wait()
        pltpu.make_async_copy(x.at[pl.ds(0, BPT)], gi.at[t], sem).wait()

    lanes = lax.iota(jnp.int32, 16)

    def grp(g_idx, carry):
        pack = jnp.zeros((16,), jnp.float32)
        for l in range(16):
            e = g_idx * 16 + l
            acc = jnp.zeros((16,), jnp.float32)
            for q in range(D // 16):
                uq = (gu[0, e, pl.ds(q * 16, 16)] + gu[1, e, pl.ds(q * 16, 16)]
                      + gu[2, e, pl.ds(q * 16, 16)]
                      + gu[3, e, pl.ds(q * 16, 16)])
                iq = (gi[0, e, pl.ds(q * 16, 16)] + gi[1, e, pl.ds(q * 16, 16)]
                      + gi[2, e, pl.ds(q * 16, 16)]
                      + gi[3, e, pl.ds(q * 16, 16)])
                acc = acc + uq * iq
            red = _lane_reduce_sum(acc, lanes) * jnp.float32(1.0 / 16.0)
            pack = jnp.where(lanes == l, red, pack)
        ob[pl.ds(g_idx * 16, 16)] = pack
        return carry
    lax.fori_loop(0, BPT // 16, grp, 0)

    pltpu.sync_copy(ob, out_hbm.at[pl.ds(base, BPT)])


def kernel(batch, A_indices, A_values, user_emb, item_emb):
    x0 = jnp.concatenate([user_emb, item_emb], axis=0)
    pad = EPADP - NEDGES
    prow = (jnp.arange(pad, dtype=jnp.int32) % NNODES).astype(jnp.int32)
    row = jnp.concatenate([A_indices[0], prow])
    col = jnp.concatenate([A_indices[1], jnp.zeros((pad,), jnp.int32)])
    val = jnp.concatenate([A_values, jnp.zeros((pad,), jnp.float32)])
    row2 = row.reshape(EPADP // 64, 64)
    col2 = col.reshape(EPADP // 64, 64)
    val2 = val.reshape(EPADP // 64, 64)

    colp, valp, lrowp = _partition(row2, col2, val2)
    colp = colp.reshape(NW, BINS, NSUB, SUB)
    valp = valp.reshape(NW, BINS, NSUB, SUB)
    lrowp = lrowp.reshape(NW, BINS, NSUB, SUB)

    x1 = _spmm(x0, colp, valp, lrowp)
    x2 = _spmm(x1, colp, valp, lrowp)
    x3 = _spmm(x2, colp, valp, lrowp)

    ui = batch[:, 0]
    ii = batch[:, 1] + NUSERS
    return _score(x0, x1, x2, x3, ui, ii)


# speedup vs baseline: 3.9771x; 3.9771x over previous
import functools
import jax
import jax.numpy as jnp
from jax import lax
from jax.experimental import pallas as pl
from jax.experimental.pallas import tpu as pltpu
from jax.experimental.pallas import tpu_sc as plsc

NUSERS = 30000
NNODES = 50000
D = 64
NEDGES = 800000
B = 4096

NC = 2
NS = 16
NW = NC * NS

HALF = NNODES // NC
TRASH = 88
ACC_ROWS = HALF + TRASH
ZROWS = ACC_ROWS // NS

SUB = 64
KF = 2
NSUB = 12
CHUNK = NSUB * SUB
BPC = NSUB // KF
NBIG = 67
PER_TILE = NBIG * CHUNK
EPAD = NS * PER_TILE
TB = NBIG * BPC
SROWS = PER_TILE // SUB

BPT = B // NW

_mesh = plsc.VectorSubcoreMesh(core_axis_name="c", subcore_axis_name="s")

_GATHER_DN = lax.GatherDimensionNumbers(
    offset_dims=(), collapsed_slice_dims=(0,), start_index_map=(0,))


def _bcast_lane(v16, lane):
    return lax.gather(v16, jnp.full((16, 1), lane, jnp.int32), _GATHER_DN,
                      slice_sizes=(1,),
                      mode=lax.GatherScatterMode.PROMISE_IN_BOUNDS)


def _shuffle(v16, idx16):
    return lax.gather(v16, idx16[:, None], _GATHER_DN, slice_sizes=(1,),
                      mode=lax.GatherScatterMode.PROMISE_IN_BOUNDS)


def _lane_reduce_sum(v16, lanes):
    for sh in (8, 4, 2, 1):
        v16 = v16 + _shuffle(v16, lanes ^ sh)
    return v16


@functools.partial(
    pl.kernel,
    out_type=jax.ShapeDtypeStruct((NNODES, D), jnp.float32),
    mesh=_mesh,
    compiler_params=pltpu.CompilerParams(use_tc_tiling_on_sc=False),
    scratch_types=[
        pltpu.VMEM((2, NSUB, SUB), jnp.int32),
        pltpu.VMEM((2, NSUB, SUB), jnp.float32),
        pltpu.VMEM((2, NSUB, SUB), jnp.int32),
        pltpu.VMEM((3 * KF, SUB, D), jnp.float32),
        pltpu.VMEM_SHARED((ACC_ROWS, D), jnp.float32),
        pltpu.SemaphoreType.DMA,
        pltpu.SemaphoreType.DMA,
        pltpu.SemaphoreType.DMA,
    ],
)
def _spmm(x_hbm, row_hbm, col_hbm, val_hbm, zeros_hbm, y_hbm,
          col_v, val_v, row_v, g_v, acc, sem_g, sem_s, sem_st):
    c = lax.axis_index("c")
    s = lax.axis_index("s")
    row_off = c * HALF
    lanes = lax.iota(jnp.int32, 16)
    trash = HALF + lax.rem(lanes * NS + s, TRASH)

    pltpu.sync_copy(zeros_hbm, acc.at[pl.ds(s * ZROWS, ZROWS)])
    plsc.subcore_barrier()

    sbase = s * SROWS

    def stage_fire(ck, parity):
        off = sbase + ck * NSUB
        pltpu.async_copy(col_hbm.at[pl.ds(off, NSUB)], col_v.at[parity],
                         sem_st)
        pltpu.async_copy(val_hbm.at[pl.ds(off, NSUB)], val_v.at[parity],
                         sem_st)
        pltpu.async_copy(row_hbm.at[pl.ds(off, NSUB)], row_v.at[parity],
                         sem_st)

    def stage_drain():
        pltpu.make_async_copy(col_hbm.at[pl.ds(0, NSUB)], col_v.at[0],
                              sem_st).wait()
        pltpu.make_async_copy(val_hbm.at[pl.ds(0, NSUB)], val_v.at[0],
                              sem_st).wait()
        pltpu.make_async_copy(row_hbm.at[pl.ds(0, NSUB)], row_v.at[0],
                              sem_st).wait()

    def lrow_pass(parity):
        def li(j, carry):
            for r in range(SUB // 16):
                rv = row_v[parity, j, pl.ds(r * 16, 16)]
                lv = rv - row_off
                ok = (lv >= 0) & (lv < HALF)
                row_v[parity, j, pl.ds(r * 16, 16)] = jnp.where(ok, lv, trash)
            return carry
        lax.fori_loop(0, NSUB, li, 0)

    def fire_gathers(t, slot_base):
        pcf = lax.rem(t // BPC, 2)
        jo = lax.rem(t, BPC) * KF
        for k in range(KF):
            pltpu.async_copy(x_hbm.at[col_v.at[pcf, jo + k]],
                             g_v.at[slot_base + k], sem_g)

    def drain_gathers(slot_base):
        for k in range(KF):
            pltpu.make_async_copy(x_hbm.at[pl.ds(0, SUB)],
                                  g_v.at[slot_base + k], sem_g).wait()

    def scale_batch(t, slot_base):
        pc = lax.rem(t // BPC, 2)
        jo = lax.rem(t, BPC) * KF
        for k in range(KF):
            def grp(gi2, carry):
                vv16 = val_v[pc, jo + k, pl.ds(gi2 * 16, 16)]
                vvs = [_bcast_lane(vv16, l) for l in range(16)]
                for l in range(16):
                    e = gi2 * 16 + l
                    for q in range(D // 16):
                        g_v[slot_base + k, e, pl.ds(q * 16, 16)] = (
                            g_v[slot_base + k, e, pl.ds(q * 16, 16)] * vvs[l])
                return carry
            lax.fori_loop(0, SUB // 16, grp, 0)

    def fire_scatters(t, slot_base):
        pc = lax.rem(t // BPC, 2)
        jo = lax.rem(t, BPC) * KF
        for k in range(KF):
            pltpu.async_copy(g_v.at[slot_base + k],
                             acc.at[row_v.at[pc, jo + k]], sem_s, add=True)

    def drain_scatters(slot_base):
        for k in range(KF):
            pltpu.make_async_copy(g_v.at[slot_base + k],
                                  acc.at[pl.ds(0, SUB)], sem_s).wait()

    def body(t, set_x, set_y, drain_sc, fire_g):
        drain_gathers(set_x)

        @pl.when((lax.rem(t, BPC) == BPC - 1) & (t < TB - 1))
        def _chunk_in():
            cin = t // BPC + 1
            stage_drain()
            lrow_pass(lax.rem(cin, 2))

        if drain_sc:
            drain_scatters(set_y)

        @pl.when(lax.rem(t, BPC) == 1)
        def _chunk_pf():
            cc = t // BPC

            @pl.when(cc + 1 <= NBIG - 1)
            def _pf():
                stage_fire(cc + 1, lax.rem(cc + 1, 2))

        if fire_g:
            fire_gathers(t + 1, set_y)
        scale_batch(t, set_x)
        fire_scatters(t, set_x)

    S0, S1, S2 = 0, KF, 2 * KF

    stage_fire(0, 0)
    stage_drain()
    lrow_pass(0)
    fire_gathers(0, S0)

    body(0, S0, S1, False, True)
    body(1, S1, S2, False, True)
    body(2, S2, S0, True, True)

    def triple(tt, carry):
        t0 = tt * 3
        body(t0, S0, S1, True, True)
        body(t0 + 1, S1, S2, True, True)
        body(t0 + 2, S2, S0, True, True)
        return carry
    lax.fori_loop(1, TB // 3 - 1, triple, 0)

    body(TB - 3, S0, S1, True, True)
    body(TB - 2, S1, S2, True, True)
    body(TB - 1, S2, S0, True, False)
    drain_scatters(S1)
    drain_scatters(S2)

    plsc.subcore_barrier()

    @pl.when(s < NS - 1)
    def _wb():
        pltpu.sync_copy(acc.at[pl.ds(s * ZROWS, ZROWS)],
                        y_hbm.at[pl.ds(row_off + s * ZROWS, ZROWS)])

    @pl.when(s == NS - 1)
    def _wb_last():
        tail = HALF - (NS - 1) * ZROWS
        pltpu.sync_copy(acc.at[pl.ds((NS - 1) * ZROWS, tail)],
                        y_hbm.at[pl.ds(row_off + (NS - 1) * ZROWS, tail)])


@functools.partial(
    pl.kernel,
    out_type=jax.ShapeDtypeStruct((B,), jnp.float32),
    mesh=_mesh,
    compiler_params=pltpu.CompilerParams(use_tc_tiling_on_sc=False),
    scratch_types=[
        pltpu.VMEM((BPT,), jnp.int32),
        pltpu.VMEM((BPT,), jnp.int32),
        pltpu.VMEM((4, BPT, D), jnp.float32),
        pltpu.VMEM((4, BPT, D), jnp.float32),
        pltpu.VMEM((BPT,), jnp.float32),
        pltpu.SemaphoreType.DMA,
    ],
)
def _score(x0, x1, x2, x3, ui_hbm, ii_hbm, out_hbm,
           ub, ib, gu, gi, ob, sem):
    c = lax.axis_index("c")
    s = lax.axis_index("s")
    w = s * NC + c
    base = w * BPT

    pltpu.sync_copy(ui_hbm.at[pl.ds(base, BPT)], ub)
    pltpu.sync_copy(ii_hbm.at[pl.ds(base, BPT)], ib)
    for t, x in enumerate((x0, x1, x2, x3)):
        pltpu.async_copy(x.at[ub], gu.at[t], sem)
        pltpu.async_copy(x.at[ib], gi.at[t], sem)
    for t, x in enumerate((x0, x1, x2, x3)):
        pltpu.make_async_copy(x.at[pl.ds(0, BPT)], gu.at[t], sem).wait()
        pltpu.make_async_copy(x.at[pl.ds(0, BPT)], gi.at[t], sem).wait()

    lanes = lax.iota(jnp.int32, 16)

    def grp(g_idx, carry):
        pack = jnp.zeros((16,), jnp.float32)
        for l in range(16):
            e = g_idx * 16 + l
            acc = jnp.zeros((16,), jnp.float32)
            for q in range(D // 16):
                uq = (gu[0, e, pl.ds(q * 16, 16)] + gu[1, e, pl.ds(q * 16, 16)]
                      + gu[2, e, pl.ds(q * 16, 16)]
                      + gu[3, e, pl.ds(q * 16, 16)])
                iq = (gi[0, e, pl.ds(q * 16, 16)] + gi[1, e, pl.ds(q * 16, 16)]
                      + gi[2, e, pl.ds(q * 16, 16)]
                      + gi[3, e, pl.ds(q * 16, 16)])
                acc = acc + uq * iq
            red = _lane_reduce_sum(acc, lanes) * jnp.float32(1.0 / 16.0)
            pack = jnp.where(lanes == l, red, pack)
        ob[pl.ds(g_idx * 16, 16)] = pack
        return carry
    lax.fori_loop(0, BPT // 16, grp, 0)

    pltpu.sync_copy(ob, out_hbm.at[pl.ds(base, BPT)])


def kernel(batch, A_indices, A_values, user_emb, item_emb):
    x0 = jnp.concatenate([user_emb, item_emb], axis=0)
    pad = EPAD - NEDGES
    row = jnp.concatenate([A_indices[0], jnp.zeros((pad,), jnp.int32)])
    col = jnp.concatenate([A_indices[1], jnp.zeros((pad,), jnp.int32)])
    val = jnp.concatenate([A_values, jnp.zeros((pad,), jnp.float32)])
    row2 = row.reshape(EPAD // SUB, SUB)
    col2 = col.reshape(EPAD // SUB, SUB)
    val2 = val.reshape(EPAD // SUB, SUB)
    zeros = jnp.zeros((ZROWS, D), jnp.float32)

    x1 = _spmm(x0, row2, col2, val2, zeros)
    x2 = _spmm(x1, row2, col2, val2, zeros)
    x3 = _spmm(x2, row2, col2, val2, zeros)

    ui = batch[:, 0]
    ii = batch[:, 1] + NUSERS
    return _score(x0, x1, x2, x3, ui, ii)
